# baseline (speedup 1.0000x reference)
import functools

import jax
import jax.numpy as jnp
from jax import lax
from jax.experimental import pallas as pl
from jax.experimental.pallas import tpu as pltpu
from jax.experimental.pallas import tpu_sc as plsc

NC = 2
NS = 16
NW = NC * NS
CHUNK = 128
ECH = 128
G = 16
D = 128
DEGW = 128

_mesh = functools.partial(
    plsc.VectorSubcoreMesh, core_axis_name="c", subcore_axis_name="s"
)


def _sc_degree(nacc, c_chunks):
  rpt = nacc // NS

  @functools.partial(
      pl.kernel,
      out_type=jax.ShapeDtypeStruct((NC, nacc, DEGW), jnp.float32),
      mesh=_mesh(),
      scratch_types=[
          pltpu.VMEM((c_chunks, ECH), jnp.int32),
          pltpu.VMEM((ECH, DEGW), jnp.float32),
          pltpu.VMEM_SHARED((nacc, DEGW), jnp.float32),
          pltpu.SemaphoreType.DMA,
      ],
  )
  def k(dst_hbm, ones_hbm, zeros_hbm, out_hbm, dst_v, ones_v, cnt_sh, sem):
    c = lax.axis_index("c")
    s = lax.axis_index("s")
    pltpu.sync_copy(dst_hbm.at[c, s], dst_v)
    pltpu.sync_copy(ones_hbm, ones_v)
    pltpu.sync_copy(zeros_hbm, cnt_sh.at[pl.ds(s * rpt, rpt)])
    plsc.subcore_barrier()

    def body(j, carry):
      pltpu.async_copy(ones_v, cnt_sh.at[dst_v.at[j]], sem, add=True)
      return carry

    lax.fori_loop(0, c_chunks, body, 0, unroll=False)

    def drain(j, carry):
      pltpu.make_async_copy(ones_hbm, ones_v, sem).wait()
      return carry

    lax.fori_loop(0, c_chunks, drain, 0, unroll=False)
    plsc.subcore_barrier()
    pltpu.sync_copy(cnt_sh.at[pl.ds(s * rpt, rpt)],
                    out_hbm.at[c, pl.ds(s * rpt, rpt)])

  return k


def _sc_spmm(nacc, c_chunks):
  rpt = nacc // NS

  nstg = c_chunks // G

  @functools.partial(
      pl.kernel,
      out_type=jax.ShapeDtypeStruct((NC, nacc, D), jnp.float32),
      mesh=_mesh(),
      scratch_types=[
          pltpu.VMEM((3, G, ECH), jnp.int32),
          pltpu.VMEM((3, G, ECH), jnp.int32),
          pltpu.VMEM((ECH, D), jnp.float32),
          pltpu.VMEM((ECH, D), jnp.float32),
          pltpu.VMEM_SHARED((nacc, D), jnp.float32),
          pltpu.SemaphoreType.DMA,
          pltpu.SemaphoreType.DMA,
          pltpu.SemaphoreType.DMA,
          pltpu.SemaphoreType.DMA,
          pltpu.SemaphoreType.DMA,
      ],
  )
  def k(h_hbm, src_hbm, dst_hbm, zrow_hbm, out_hbm,
        src_v, dst_v, rows_a, rows_b, acc_sh, sem_a, sem_b, ssa, ssb, isem):
    c = lax.axis_index("c")
    s = lax.axis_index("s")
    pltpu.sync_copy(src_hbm.at[c, s, pl.ds(0, G)], src_v.at[0])
    pltpu.sync_copy(dst_hbm.at[c, s, pl.ds(0, G)], dst_v.at[0])
    pltpu.async_copy(src_hbm.at[c, s, pl.ds(G, G)], src_v.at[1], isem)
    pltpu.async_copy(dst_hbm.at[c, s, pl.ds(G, G)], dst_v.at[1], isem)
    pltpu.sync_copy(zrow_hbm, rows_a)
    for kk in range(rpt // ECH):
      pltpu.sync_copy(rows_a, acc_sh.at[pl.ds(s * rpt + kk * ECH, ECH)])
    plsc.subcore_barrier()

    def wait(buf, sem):
      pltpu.make_async_copy(h_hbm.at[pl.ds(0, ECH)], buf, sem).wait()

    def src_row(j):
      return src_v.at[(j // G) % 3, j % G]

    def dst_row(j):
      return dst_v.at[(j // G) % 3, j % G]

    def fire(j, buf, sem):
      @pl.when(jnp.logical_and(j % G == 0, j > 0))
      def _():
        pltpu.make_async_copy(src_hbm.at[0, 0, pl.ds(0, G)], src_v.at[0],
                              isem).wait()
        pltpu.make_async_copy(dst_hbm.at[0, 0, pl.ds(0, G)], dst_v.at[0],
                              isem).wait()
        st1 = j // G + 1

        @pl.when(st1 < nstg)
        def _():
          pltpu.async_copy(src_hbm.at[c, s, pl.ds(st1 * G, G)],
                           src_v.at[st1 % 3], isem)
          pltpu.async_copy(dst_hbm.at[c, s, pl.ds(st1 * G, G)],
                           dst_v.at[st1 % 3], isem)

      pltpu.async_copy(h_hbm.at[src_row(j)], buf, sem)

    fire(0, rows_a, sem_a)

    def body(jj, carry):
      j0 = 2 * jj
      j1 = j0 + 1
      j2 = j0 + 2
      wait(rows_a, sem_a)

      @pl.when(jj > 0)
      def _():
        wait(rows_b, ssb)
      fire(j1, rows_b, sem_b)
      pltpu.async_copy(rows_a, acc_sh.at[dst_row(j0)], ssa, add=True)
      wait(rows_b, sem_b)
      pltpu.async_copy(rows_b, acc_sh.at[dst_row(j1)], ssb, add=True)
      wait(rows_a, ssa)

      @pl.when(j2 < c_chunks)
      def _():
        fire(j2, rows_a, sem_a)
      return carry

    lax.fori_loop(0, c_chunks // 2, body, 0, unroll=False)
    wait(rows_b, ssb)
    plsc.subcore_barrier()
    pltpu.sync_copy(acc_sh.at[pl.ds(s * rpt, rpt)],
                    out_hbm.at[c, pl.ds(s * rpt, rpt)])

  return k


def _tc_first(nacc):
  nb = nacc // CHUNK

  def body(d0, d1, x, w, dinv_ref, h_ref):
    deg = d0[:, 0:1] + d1[:, 0:1] + 1.0
    dinv = lax.rsqrt(deg)
    g = jnp.dot(x[...], w[...], preferred_element_type=jnp.float32)
    dinv_ref[...] = dinv
    h_ref[...] = dinv * g

  return pl.pallas_call(
      body,
      grid=(nb,),
      in_specs=[
          pl.BlockSpec((CHUNK, DEGW), lambda i: (i, 0)),
          pl.BlockSpec((CHUNK, DEGW), lambda i: (i, 0)),
          pl.BlockSpec((CHUNK, D), lambda i: (i, 0)),
          pl.BlockSpec((D, D), lambda i: (0, 0)),
      ],
      out_specs=[
          pl.BlockSpec((CHUNK, 1), lambda i: (i, 0)),
          pl.BlockSpec((CHUNK, D), lambda i: (i, 0)),
      ],
      out_shape=[
          jax.ShapeDtypeStruct((nacc, 1), jnp.float32),
          jax.ShapeDtypeStruct((nacc, D), jnp.float32),
      ],
  )


def _tc_mid(nacc):
  nb = nacc // CHUNK

  def body(a0, a1, hp, dinv, b, w, o_ref, h_ref):
    acc = a0[...] + a1[...] + hp[...]
    o = jnp.maximum(dinv[...] * acc + b[...], 0.0)
    o_ref[...] = o
    h_ref[...] = dinv[...] * jnp.dot(
        o, w[...], preferred_element_type=jnp.float32)

  return pl.pallas_call(
      body,
      grid=(nb,),
      in_specs=[
          pl.BlockSpec((CHUNK, D), lambda i: (i, 0)),
          pl.BlockSpec((CHUNK, D), lambda i: (i, 0)),
          pl.BlockSpec((CHUNK, D), lambda i: (i, 0)),
          pl.BlockSpec((CHUNK, 1), lambda i: (i, 0)),
          pl.BlockSpec((1, D), lambda i: (0, 0)),
          pl.BlockSpec((D, D), lambda i: (0, 0)),
      ],
      out_specs=[
          pl.BlockSpec((CHUNK, D), lambda i: (i, 0)),
          pl.BlockSpec((CHUNK, D), lambda i: (i, 0)),
      ],
      out_shape=[
          jax.ShapeDtypeStruct((nacc, D), jnp.float32),
          jax.ShapeDtypeStruct((nacc, D), jnp.float32),
      ],
  )


def _tc_last(nacc):
  nb = nacc // CHUNK

  def body(a0, a1, hp, dinv, b, o0, o1, o2, wo, bo, out_ref):
    acc = a0[...] + a1[...] + hp[...]
    o3 = jnp.maximum(dinv[...] * acc + b[...], 0.0)
    w = wo[...]
    r = jnp.dot(o0[...], w[0:128], preferred_element_type=jnp.float32)
    r += jnp.dot(o1[...], w[128:256], preferred_element_type=jnp.float32)
    r += jnp.dot(o2[...], w[256:384], preferred_element_type=jnp.float32)
    r += jnp.dot(o3, w[384:512], preferred_element_type=jnp.float32)
    out_ref[...] = r + bo[...]

  return pl.pallas_call(
      body,
      grid=(nb,),
      in_specs=[
          pl.BlockSpec((CHUNK, D), lambda i: (i, 0)),
          pl.BlockSpec((CHUNK, D), lambda i: (i, 0)),
          pl.BlockSpec((CHUNK, D), lambda i: (i, 0)),
          pl.BlockSpec((CHUNK, 1), lambda i: (i, 0)),
          pl.BlockSpec((1, D), lambda i: (0, 0)),
          pl.BlockSpec((CHUNK, D), lambda i: (i, 0)),
          pl.BlockSpec((CHUNK, D), lambda i: (i, 0)),
          pl.BlockSpec((CHUNK, D), lambda i: (i, 0)),
          pl.BlockSpec((4 * D, D), lambda i: (0, 0)),
          pl.BlockSpec((1, D), lambda i: (0, 0)),
      ],
      out_specs=pl.BlockSpec((CHUNK, D), lambda i: (i, 0)),
      out_shape=jax.ShapeDtypeStruct((nacc, D), jnp.float32),
  )


def kernel(x, edge_index, W0, b0, W1, b1, W2, b2, W3, b3, Wout, bout):
  n = x.shape[0]
  e = edge_index.shape[1]

  nacc = ((n + NS * CHUNK - 1) // (NS * CHUNK)) * NS * CHUNK
  pad_rows = nacc - n

  ee = NW * ECH * G
  ep = ((e + ee - 1) // ee) * ee
  pad_e = ep - e
  c_chunks = ep // (NW * ECH)
  pad_idx = n + (jnp.arange(pad_e, dtype=jnp.int32) % jnp.int32(pad_rows))
  src = jnp.concatenate([edge_index[0], pad_idx]).reshape(NC, NS, c_chunks,
                                                          ECH)
  dst = jnp.concatenate([edge_index[1], pad_idx]).reshape(NC, NS, c_chunks,
                                                          ECH)

  ones_deg = jnp.ones((ECH, DEGW), jnp.float32)
  zeros_deg = jnp.zeros((nacc // NS, DEGW), jnp.float32)
  zrow = jnp.zeros((ECH, D), jnp.float32)
  x_pad = jnp.pad(x, ((0, pad_rows), (0, 0)))

  deg_k = _sc_degree(nacc, c_chunks)
  spmm_k = _sc_spmm(nacc, c_chunks)
  tc_first = _tc_first(nacc)
  tc_mid = _tc_mid(nacc)
  tc_last = _tc_last(nacc)

  cnt = deg_k(dst, ones_deg, zeros_deg)
  dinv, hp = tc_first(cnt[0], cnt[1], x_pad, W0)

  acc = spmm_k(hp, src, dst, zrow)
  o0, hp = tc_mid(acc[0], acc[1], hp, dinv, b0.reshape(1, D), W1)
  acc = spmm_k(hp, src, dst, zrow)
  o1, hp = tc_mid(acc[0], acc[1], hp, dinv, b1.reshape(1, D), W2)
  acc = spmm_k(hp, src, dst, zrow)
  o2, hp = tc_mid(acc[0], acc[1], hp, dinv, b2.reshape(1, D), W3)
  acc = spmm_k(hp, src, dst, zrow)
  out = tc_last(acc[0], acc[1], hp, dinv, b3.reshape(1, D), o0, o1, o2,
                Wout, bout.reshape(1, D))
  return out[:n]


# speedup vs baseline: 1.4824x; 1.4824x over previous
import functools

import jax
import jax.numpy as jnp
import numpy as np
from jax import lax
from jax.experimental import pallas as pl
from jax.experimental.pallas import tpu as pltpu
from jax.experimental.pallas import tpu_sc as plsc

NC = 2
NS = 16
NW = NC * NS
CHUNK = 128
ECH = 128
G = 16
D = 128
TBLK = 2048
DEGW = 128

_mesh = functools.partial(
    plsc.VectorSubcoreMesh, core_axis_name="c", subcore_axis_name="s"
)


def _sc_degree(nacc, c_chunks):
  rpt = nacc // NS

  @functools.partial(
      pl.kernel,
      out_type=jax.ShapeDtypeStruct((NC, nacc, DEGW), jnp.float32),
      mesh=_mesh(),
      scratch_types=[
          pltpu.VMEM((c_chunks, ECH), jnp.int32),
          pltpu.VMEM((ECH, DEGW), jnp.float32),
          pltpu.VMEM_SHARED((nacc, DEGW), jnp.float32),
          pltpu.SemaphoreType.DMA,
      ],
  )
  def k(dst_hbm, ones_hbm, zeros_hbm, out_hbm, dst_v, ones_v, cnt_sh, sem):
    c = lax.axis_index("c")
    s = lax.axis_index("s")
    pltpu.sync_copy(dst_hbm.at[c, s], dst_v)
    pltpu.sync_copy(ones_hbm, ones_v)
    pltpu.sync_copy(zeros_hbm, cnt_sh.at[pl.ds(s * rpt, rpt)])
    plsc.subcore_barrier()

    def body(j, carry):
      pltpu.async_copy(ones_v, cnt_sh.at[dst_v.at[j]], sem, add=True)
      return carry

    lax.fori_loop(0, c_chunks, body, 0, unroll=False)

    def drain(j, carry):
      pltpu.make_async_copy(ones_hbm, ones_v, sem).wait()
      return carry

    lax.fori_loop(0, c_chunks, drain, 0, unroll=False)
    plsc.subcore_barrier()
    pltpu.sync_copy(cnt_sh.at[pl.ds(s * rpt, rpt)],
                    out_hbm.at[c, pl.ds(s * rpt, rpt)])

  return k


def _sc_spmm(nacc, c_chunks):
  rpt = nacc // NS

  nstg = c_chunks // G

  @functools.partial(
      pl.kernel,
      out_type=jax.ShapeDtypeStruct((NC, nacc, D), jnp.float32),
      mesh=_mesh(),
      scratch_types=[
          pltpu.VMEM((3, G, ECH), jnp.int32),
          pltpu.VMEM((3, G, ECH), jnp.int32),
          pltpu.VMEM((ECH, D), jnp.float32),
          pltpu.VMEM((ECH, D), jnp.float32),
          pltpu.VMEM_SHARED((nacc, D), jnp.float32),
          pltpu.SemaphoreType.DMA,
          pltpu.SemaphoreType.DMA,
          pltpu.SemaphoreType.DMA,
          pltpu.SemaphoreType.DMA,
          pltpu.SemaphoreType.DMA,
      ],
  )
  def k(h_hbm, src_hbm, dst_hbm, zrow_hbm, out_hbm,
        src_v, dst_v, rows_a, rows_b, acc_sh, sem_a, sem_b, ssa, ssb, isem):
    c = lax.axis_index("c")
    s = lax.axis_index("s")
    pltpu.sync_copy(src_hbm.at[c, s, pl.ds(0, G)], src_v.at[0])
    pltpu.sync_copy(dst_hbm.at[c, s, pl.ds(0, G)], dst_v.at[0])
    pltpu.async_copy(src_hbm.at[c, s, pl.ds(G, G)], src_v.at[1], isem)
    pltpu.async_copy(dst_hbm.at[c, s, pl.ds(G, G)], dst_v.at[1], isem)
    pltpu.sync_copy(zrow_hbm, rows_a)
    for kk in range(rpt // ECH):
      pltpu.sync_copy(rows_a, acc_sh.at[pl.ds(s * rpt + kk * ECH, ECH)])
    plsc.subcore_barrier()

    def wait(buf, sem):
      pltpu.make_async_copy(h_hbm.at[pl.ds(0, ECH)], buf, sem).wait()

    def src_row(j):
      return src_v.at[(j // G) % 3, j % G]

    def dst_row(j):
      return dst_v.at[(j // G) % 3, j % G]

    def fire(j, buf, sem):
      @pl.when(jnp.logical_and(j % G == 0, j > 0))
      def _():
        pltpu.make_async_copy(src_hbm.at[0, 0, pl.ds(0, G)], src_v.at[0],
                              isem).wait()
        pltpu.make_async_copy(dst_hbm.at[0, 0, pl.ds(0, G)], dst_v.at[0],
                              isem).wait()
        st1 = j // G + 1

        @pl.when(st1 < nstg)
        def _():
          pltpu.async_copy(src_hbm.at[c, s, pl.ds(st1 * G, G)],
                           src_v.at[st1 % 3], isem)
          pltpu.async_copy(dst_hbm.at[c, s, pl.ds(st1 * G, G)],
                           dst_v.at[st1 % 3], isem)

      pltpu.async_copy(h_hbm.at[src_row(j)], buf, sem)

    fire(0, rows_a, sem_a)

    def body(jj, carry):
      j0 = 2 * jj
      j1 = j0 + 1
      j2 = j0 + 2
      wait(rows_a, sem_a)
      fire(j1, rows_b, sem_b)
      pltpu.sync_copy(rows_a, acc_sh.at[dst_row(j0)], add=True)

      @pl.when(j2 < c_chunks)
      def _():
        fire(j2, rows_a, sem_a)

      wait(rows_b, sem_b)
      pltpu.sync_copy(rows_b, acc_sh.at[dst_row(j1)], add=True)
      return carry

    lax.fori_loop(0, c_chunks // 2, body, 0, unroll=False)
    plsc.subcore_barrier()
    pltpu.sync_copy(acc_sh.at[pl.ds(s * rpt, rpt)],
                    out_hbm.at[c, pl.ds(s * rpt, rpt)])

  return k


def _tc_first(nacc):
  nb = nacc // TBLK

  def body(cnt, x, w, dinv_ref, h_ref):
    deg = cnt[0, :, 0:1] + cnt[1, :, 0:1] + 1.0
    dinv = lax.rsqrt(deg)
    g = jnp.dot(x[...], w[...], preferred_element_type=jnp.float32)
    dinv_ref[...] = dinv
    h_ref[...] = dinv * g

  return pl.pallas_call(
      body,
      grid=(nb,),
      in_specs=[
          pl.BlockSpec((2, TBLK, DEGW), lambda i: (0, i, 0)),
          pl.BlockSpec((TBLK, D), lambda i: (i, 0)),
          pl.BlockSpec((D, D), lambda i: (0, 0)),
      ],
      out_specs=[
          pl.BlockSpec((TBLK, 1), lambda i: (i, 0)),
          pl.BlockSpec((TBLK, D), lambda i: (i, 0)),
      ],
      out_shape=[
          jax.ShapeDtypeStruct((nacc, 1), jnp.float32),
          jax.ShapeDtypeStruct((nacc, D), jnp.float32),
      ],
  )


def _tc_mid(nacc):
  nb = nacc // TBLK

  def body(a, hp, dinv, b, w, o_ref, h_ref):
    acc = a[0] + a[1] + hp[...]
    o = jnp.maximum(dinv[...] * acc + b[...], 0.0)
    o_ref[...] = o
    h_ref[...] = dinv[...] * jnp.dot(
        o, w[...], preferred_element_type=jnp.float32)

  return pl.pallas_call(
      body,
      grid=(nb,),
      in_specs=[
          pl.BlockSpec((2, TBLK, D), lambda i: (0, i, 0)),
          pl.BlockSpec((TBLK, D), lambda i: (i, 0)),
          pl.BlockSpec((TBLK, 1), lambda i: (i, 0)),
          pl.BlockSpec((1, D), lambda i: (0, 0)),
          pl.BlockSpec((D, D), lambda i: (0, 0)),
      ],
      out_specs=[
          pl.BlockSpec((TBLK, D), lambda i: (i, 0)),
          pl.BlockSpec((TBLK, D), lambda i: (i, 0)),
      ],
      out_shape=[
          jax.ShapeDtypeStruct((nacc, D), jnp.float32),
          jax.ShapeDtypeStruct((nacc, D), jnp.float32),
      ],
  )


def _tc_last(nacc):
  nb = nacc // TBLK

  def body(a, hp, dinv, b, o0, o1, o2, wo, bo, out_ref):
    acc = a[0] + a[1] + hp[...]
    o3 = jnp.maximum(dinv[...] * acc + b[...], 0.0)
    w = wo[...]
    r = jnp.dot(o0[...], w[0:128], preferred_element_type=jnp.float32)
    r += jnp.dot(o1[...], w[128:256], preferred_element_type=jnp.float32)
    r += jnp.dot(o2[...], w[256:384], preferred_element_type=jnp.float32)
    r += jnp.dot(o3, w[384:512], preferred_element_type=jnp.float32)
    out_ref[...] = r + bo[...]

  return pl.pallas_call(
      body,
      grid=(nb,),
      in_specs=[
          pl.BlockSpec((2, TBLK, D), lambda i: (0, i, 0)),
          pl.BlockSpec((TBLK, D), lambda i: (i, 0)),
          pl.BlockSpec((TBLK, 1), lambda i: (i, 0)),
          pl.BlockSpec((1, D), lambda i: (0, 0)),
          pl.BlockSpec((TBLK, D), lambda i: (i, 0)),
          pl.BlockSpec((TBLK, D), lambda i: (i, 0)),
          pl.BlockSpec((TBLK, D), lambda i: (i, 0)),
          pl.BlockSpec((4 * D, D), lambda i: (0, 0)),
          pl.BlockSpec((1, D), lambda i: (0, 0)),
      ],
      out_specs=pl.BlockSpec((TBLK, D), lambda i: (i, 0)),
      out_shape=jax.ShapeDtypeStruct((nacc, D), jnp.float32),
  )


def kernel(x, edge_index, W0, b0, W1, b1, W2, b2, W3, b3, Wout, bout):
  n = x.shape[0]
  e = edge_index.shape[1]

  nacc = ((n + NS * CHUNK - 1) // (NS * CHUNK)) * NS * CHUNK
  pad_rows = nacc - n

  ee = NW * ECH * G
  ep = ((e + ee - 1) // ee) * ee
  pad_e = ep - e
  c_chunks = ep // (NW * ECH)
  pad_idx = jnp.asarray(n + (np.arange(pad_e) % pad_rows), dtype=jnp.int32)
  src = jnp.concatenate([edge_index[0], pad_idx]).reshape(NC, NS, c_chunks,
                                                          ECH)
  dst = jnp.concatenate([edge_index[1], pad_idx]).reshape(NC, NS, c_chunks,
                                                          ECH)

  ones_deg = jnp.ones((ECH, DEGW), jnp.float32)
  zeros_deg = jnp.zeros((nacc // NS, DEGW), jnp.float32)
  zrow = jnp.zeros((ECH, D), jnp.float32)
  x_pad = jnp.pad(x, ((0, pad_rows), (0, 0)))

  deg_k = _sc_degree(nacc, c_chunks)
  spmm_k = _sc_spmm(nacc, c_chunks)
  tc_first = _tc_first(nacc)
  tc_mid = _tc_mid(nacc)
  tc_last = _tc_last(nacc)

  cnt = deg_k(dst, ones_deg, zeros_deg)
  dinv, hp = tc_first(cnt, x_pad, W0)

  acc = spmm_k(hp, src, dst, zrow)
  o0, hp = tc_mid(acc, hp, dinv, b0.reshape(1, D), W1)
  acc = spmm_k(hp, src, dst, zrow)
  o1, hp = tc_mid(acc, hp, dinv, b1.reshape(1, D), W2)
  acc = spmm_k(hp, src, dst, zrow)
  o2, hp = tc_mid(acc, hp, dinv, b2.reshape(1, D), W3)
  acc = spmm_k(hp, src, dst, zrow)
  out = tc_last(acc, hp, dinv, b3.reshape(1, D), o0, o1, o2,
                Wout, bout.reshape(1, D))
  return out[:n]


# speedup vs baseline: 1.4958x; 1.0091x over previous
import functools

import jax
import jax.numpy as jnp
import numpy as np
from jax import lax
from jax.experimental import pallas as pl
from jax.experimental.pallas import tpu as pltpu
from jax.experimental.pallas import tpu_sc as plsc

NC = 2
NS = 16
NW = NC * NS
CHUNK = 128
ECH = 128
G = 16
D = 128
TBLK = 2048
DEGW = 128

_mesh = functools.partial(
    plsc.VectorSubcoreMesh, core_axis_name="c", subcore_axis_name="s"
)


def _sc_degree(nacc, c_chunks):
  rpt = nacc // NS

  @functools.partial(
      pl.kernel,
      out_type=jax.ShapeDtypeStruct((NC, nacc, DEGW), jnp.float32),
      mesh=_mesh(),
      scratch_types=[
          pltpu.VMEM((c_chunks, ECH), jnp.int32),
          pltpu.VMEM((ECH, DEGW), jnp.float32),
          pltpu.VMEM_SHARED((nacc, DEGW), jnp.float32),
          pltpu.SemaphoreType.DMA,
      ],
  )
  def k(dst_hbm, ones_hbm, zeros_hbm, out_hbm, dst_v, ones_v, cnt_sh, sem):
    c = lax.axis_index("c")
    s = lax.axis_index("s")
    pltpu.sync_copy(dst_hbm.at[c, s], dst_v)
    pltpu.sync_copy(ones_hbm, ones_v)
    pltpu.sync_copy(zeros_hbm, cnt_sh.at[pl.ds(s * rpt, rpt)])
    plsc.subcore_barrier()

    def body(j, carry):
      pltpu.async_copy(ones_v, cnt_sh.at[dst_v.at[j]], sem, add=True)
      return carry

    lax.fori_loop(0, c_chunks, body, 0, unroll=False)

    def drain(j, carry):
      pltpu.make_async_copy(ones_hbm, ones_v, sem).wait()
      return carry

    lax.fori_loop(0, c_chunks, drain, 0, unroll=False)
    plsc.subcore_barrier()
    pltpu.sync_copy(cnt_sh.at[pl.ds(s * rpt, rpt)],
                    out_hbm.at[c, pl.ds(s * rpt, rpt)])

  return k


def _sc_spmm(nacc, c_chunks):
  rpt = nacc // NS

  nstg = c_chunks // G

  @functools.partial(
      pl.kernel,
      out_type=jax.ShapeDtypeStruct((NC, nacc, D), jnp.float32),
      mesh=_mesh(),
      scratch_types=[
          pltpu.VMEM((3, G, ECH), jnp.int32),
          pltpu.VMEM((3, G, ECH), jnp.int32),
          pltpu.VMEM((ECH, D), jnp.float32),
          pltpu.VMEM((ECH, D), jnp.float32),
          pltpu.VMEM_SHARED((nacc, D), jnp.float32),
          pltpu.SemaphoreType.DMA,
          pltpu.SemaphoreType.DMA,
          pltpu.SemaphoreType.DMA,
          pltpu.SemaphoreType.DMA,
          pltpu.SemaphoreType.DMA,
      ],
  )
  def k(h_hbm, src_hbm, dst_hbm, zrow_hbm, out_hbm,
        src_v, dst_v, rows_a, rows_b, acc_sh, sem_a, sem_b, ssa, ssb, isem):
    c = lax.axis_index("c")
    s = lax.axis_index("s")
    pltpu.sync_copy(src_hbm.at[c, s, pl.ds(0, G)], src_v.at[0])
    pltpu.sync_copy(dst_hbm.at[c, s, pl.ds(0, G)], dst_v.at[0])
    pltpu.async_copy(src_hbm.at[c, s, pl.ds(G, G)], src_v.at[1], isem)
    pltpu.async_copy(dst_hbm.at[c, s, pl.ds(G, G)], dst_v.at[1], isem)
    pltpu.sync_copy(zrow_hbm, rows_a)
    for kk in range(rpt // ECH):
      pltpu.sync_copy(rows_a, acc_sh.at[pl.ds(s * rpt + kk * ECH, ECH)])
    plsc.subcore_barrier()

    def wait(buf, sem):
      pltpu.make_async_copy(h_hbm.at[pl.ds(0, ECH)], buf, sem).wait()

    def src_row(j):
      return src_v.at[(j // G) % 3, j % G]

    def dst_row(j):
      return dst_v.at[(j // G) % 3, j % G]

    def fire(j, buf, sem):
      @pl.when(jnp.logical_and(j % G == 0, j > 0))
      def _():
        pltpu.make_async_copy(src_hbm.at[0, 0, pl.ds(0, G)], src_v.at[0],
                              isem).wait()
        pltpu.make_async_copy(dst_hbm.at[0, 0, pl.ds(0, G)], dst_v.at[0],
                              isem).wait()
        st1 = j // G + 1

        @pl.when(st1 < nstg)
        def _():
          pltpu.async_copy(src_hbm.at[c, s, pl.ds(st1 * G, G)],
                           src_v.at[st1 % 3], isem)
          pltpu.async_copy(dst_hbm.at[c, s, pl.ds(st1 * G, G)],
                           dst_v.at[st1 % 3], isem)

      pltpu.async_copy(h_hbm.at[src_row(j)], buf, sem)

    fire(0, rows_a, sem_a)

    def body(jj, carry):
      j0 = 2 * jj
      j1 = j0 + 1
      j2 = j0 + 2
      wait(rows_a, sem_a)
      fire(j1, rows_b, sem_b)
      pltpu.sync_copy(rows_a, acc_sh.at[dst_row(j0)], add=True)

      @pl.when(j2 < c_chunks)
      def _():
        fire(j2, rows_a, sem_a)

      wait(rows_b, sem_b)
      pltpu.sync_copy(rows_b, acc_sh.at[dst_row(j1)], add=True)
      return carry

    lax.fori_loop(0, c_chunks // 2, body, 0, unroll=False)
    plsc.subcore_barrier()
    pltpu.sync_copy(acc_sh.at[pl.ds(s * rpt, rpt)],
                    out_hbm.at[c, pl.ds(s * rpt, rpt)])

  return k


def _tc_w0(nacc):
  nb = nacc // TBLK

  def body(x, w, g_ref):
    g_ref[...] = jnp.dot(x[...], w[...], preferred_element_type=jnp.float32)

  return pl.pallas_call(
      body,
      grid=(nb,),
      in_specs=[
          pl.BlockSpec((TBLK, D), lambda i: (i, 0)),
          pl.BlockSpec((D, D), lambda i: (0, 0)),
      ],
      out_specs=pl.BlockSpec((TBLK, D), lambda i: (i, 0)),
      out_shape=jax.ShapeDtypeStruct((nacc, D), jnp.float32),
  )


def _tc_first(nacc):
  nb = nacc // TBLK

  def body(cnt, g, dinv_ref, h_ref):
    deg = cnt[0, :, 0:1] + cnt[1, :, 0:1] + 1.0
    dinv = lax.rsqrt(deg)
    dinv_ref[...] = dinv
    h_ref[...] = dinv * g[...]

  return pl.pallas_call(
      body,
      grid=(nb,),
      in_specs=[
          pl.BlockSpec((2, TBLK, DEGW), lambda i: (0, i, 0)),
          pl.BlockSpec((TBLK, D), lambda i: (i, 0)),
      ],
      out_specs=[
          pl.BlockSpec((TBLK, 1), lambda i: (i, 0)),
          pl.BlockSpec((TBLK, D), lambda i: (i, 0)),
      ],
      out_shape=[
          jax.ShapeDtypeStruct((nacc, 1), jnp.float32),
          jax.ShapeDtypeStruct((nacc, D), jnp.float32),
      ],
  )


def _tc_mid(nacc):
  nb = nacc // TBLK

  def body(a, hp, dinv, b, w, o_ref, h_ref):
    acc = a[0] + a[1] + hp[...]
    o = jnp.maximum(dinv[...] * acc + b[...], 0.0)
    o_ref[...] = o
    h_ref[...] = dinv[...] * jnp.dot(
        o, w[...], preferred_element_type=jnp.float32)

  return pl.pallas_call(
      body,
      grid=(nb,),
      in_specs=[
          pl.BlockSpec((2, TBLK, D), lambda i: (0, i, 0)),
          pl.BlockSpec((TBLK, D), lambda i: (i, 0)),
          pl.BlockSpec((TBLK, 1), lambda i: (i, 0)),
          pl.BlockSpec((1, D), lambda i: (0, 0)),
          pl.BlockSpec((D, D), lambda i: (0, 0)),
      ],
      out_specs=[
          pl.BlockSpec((TBLK, D), lambda i: (i, 0)),
          pl.BlockSpec((TBLK, D), lambda i: (i, 0)),
      ],
      out_shape=[
          jax.ShapeDtypeStruct((nacc, D), jnp.float32),
          jax.ShapeDtypeStruct((nacc, D), jnp.float32),
      ],
  )


def _tc_jk_partial(nacc):
  nb = nacc // TBLK

  def body(o0, o1, o2, wo, bo, out_ref):
    w = wo[...]
    r = jnp.dot(o0[...], w[0:128], preferred_element_type=jnp.float32)
    r += jnp.dot(o1[...], w[128:256], preferred_element_type=jnp.float32)
    r += jnp.dot(o2[...], w[256:384], preferred_element_type=jnp.float32)
    out_ref[...] = r + bo[...]

  return pl.pallas_call(
      body,
      grid=(nb,),
      in_specs=[
          pl.BlockSpec((TBLK, D), lambda i: (i, 0)),
          pl.BlockSpec((TBLK, D), lambda i: (i, 0)),
          pl.BlockSpec((TBLK, D), lambda i: (i, 0)),
          pl.BlockSpec((4 * D, D), lambda i: (0, 0)),
          pl.BlockSpec((1, D), lambda i: (0, 0)),
      ],
      out_specs=pl.BlockSpec((TBLK, D), lambda i: (i, 0)),
      out_shape=jax.ShapeDtypeStruct((nacc, D), jnp.float32),
  )


def _tc_last(nacc, n):
  nb = nacc // TBLK

  def body(a, hp, dinv, b, jkp, wo, out_ref):
    acc = a[0] + a[1] + hp[...]
    o3 = jnp.maximum(dinv[...] * acc + b[...], 0.0)
    w = wo[...]
    out_ref[...] = jkp[...] + jnp.dot(o3, w[384:512],
                                      preferred_element_type=jnp.float32)

  return pl.pallas_call(
      body,
      grid=(nb,),
      in_specs=[
          pl.BlockSpec((2, TBLK, D), lambda i: (0, i, 0)),
          pl.BlockSpec((TBLK, D), lambda i: (i, 0)),
          pl.BlockSpec((TBLK, 1), lambda i: (i, 0)),
          pl.BlockSpec((1, D), lambda i: (0, 0)),
          pl.BlockSpec((TBLK, D), lambda i: (i, 0)),
          pl.BlockSpec((4 * D, D), lambda i: (0, 0)),
      ],
      out_specs=pl.BlockSpec((TBLK, D), lambda i: (i, 0)),
      out_shape=jax.ShapeDtypeStruct((n, D), jnp.float32),
  )


def kernel(x, edge_index, W0, b0, W1, b1, W2, b2, W3, b3, Wout, bout):
  n = x.shape[0]
  e = edge_index.shape[1]

  nacc = ((n + NS * CHUNK - 1) // (NS * CHUNK)) * NS * CHUNK
  pad_rows = nacc - n

  ee = NW * ECH * G
  ep = ((e + ee - 1) // ee) * ee
  pad_e = ep - e
  c_chunks = ep // (NW * ECH)
  pad_idx = jnp.asarray(n + (np.arange(pad_e) % pad_rows), dtype=jnp.int32)
  src = jnp.concatenate([edge_index[0], pad_idx]).reshape(NC, NS, c_chunks,
                                                          ECH)
  dst = jnp.concatenate([edge_index[1], pad_idx]).reshape(NC, NS, c_chunks,
                                                          ECH)

  ones_deg = jnp.ones((ECH, DEGW), jnp.float32)
  zeros_deg = jnp.zeros((nacc // NS, DEGW), jnp.float32)
  zrow = jnp.zeros((ECH, D), jnp.float32)
  x_pad = jnp.pad(x, ((0, pad_rows), (0, 0)))

  deg_k = _sc_degree(nacc, c_chunks)
  spmm_k = _sc_spmm(nacc, c_chunks)
  tc_w0 = _tc_w0(nacc)
  tc_first = _tc_first(nacc)
  tc_mid = _tc_mid(nacc)
  tc_jkp = _tc_jk_partial(nacc)
  tc_last = _tc_last(nacc, n)

  cnt = deg_k(dst, ones_deg, zeros_deg)
  g0 = tc_w0(x_pad, W0)
  dinv, hp = tc_first(cnt, g0)

  acc = spmm_k(hp, src, dst, zrow)
  o0, hp = tc_mid(acc, hp, dinv, b0.reshape(1, D), W1)
  acc = spmm_k(hp, src, dst, zrow)
  o1, hp = tc_mid(acc, hp, dinv, b1.reshape(1, D), W2)
  acc = spmm_k(hp, src, dst, zrow)
  o2, hp = tc_mid(acc, hp, dinv, b2.reshape(1, D), W3)
  acc = spmm_k(hp, src, dst, zrow)
  jkp = tc_jkp(o0, o1, o2, Wout, bout.reshape(1, D))
  out = tc_last(acc, hp, dinv, b3.reshape(1, D), jkp, Wout)
  return out


# speedup vs baseline: 1.5195x; 1.0158x over previous
import functools

import jax
import jax.numpy as jnp
import numpy as np
from jax import lax
from jax.experimental import pallas as pl
from jax.experimental.pallas import tpu as pltpu
from jax.experimental.pallas import tpu_sc as plsc

NC = 2
NS = 16
NW = NC * NS
CHUNK = 128
ECH = 128
G = 16
D = 128
TBLK = 2048
DEGW = 128

_mesh = functools.partial(
    plsc.VectorSubcoreMesh, core_axis_name="c", subcore_axis_name="s"
)


def _sc_degree(nacc, c_chunks):
  rpt = nacc // NS

  @functools.partial(
      pl.kernel,
      out_type=jax.ShapeDtypeStruct((NC, nacc, DEGW), jnp.float32),
      mesh=_mesh(),
      scratch_types=[
          pltpu.VMEM((c_chunks, ECH), jnp.int32),
          pltpu.VMEM((ECH, DEGW), jnp.float32),
          pltpu.VMEM_SHARED((nacc, DEGW), jnp.float32),
          pltpu.SemaphoreType.DMA,
      ],
  )
  def k(e_hbm, ones_hbm, zeros_hbm, out_hbm, dst_v, ones_v, cnt_sh, sem):
    c = lax.axis_index("c")
    s = lax.axis_index("s")
    pltpu.sync_copy(e_hbm.at[1, c, s], dst_v)
    pltpu.sync_copy(ones_hbm, ones_v)
    pltpu.sync_copy(zeros_hbm, cnt_sh.at[pl.ds(s * rpt, rpt)])
    plsc.subcore_barrier()

    def body(j, carry):
      pltpu.async_copy(ones_v, cnt_sh.at[dst_v.at[j]], sem, add=True)
      return carry

    lax.fori_loop(0, c_chunks, body, 0, unroll=False)

    def drain(j, carry):
      pltpu.make_async_copy(ones_hbm, ones_v, sem).wait()
      return carry

    lax.fori_loop(0, c_chunks, drain, 0, unroll=False)
    plsc.subcore_barrier()
    pltpu.sync_copy(cnt_sh.at[pl.ds(s * rpt, rpt)],
                    out_hbm.at[c, pl.ds(s * rpt, rpt)])

  return k


def _sc_spmm(nacc, c_chunks):
  rpt = nacc // NS

  nstg = c_chunks // G

  @functools.partial(
      pl.kernel,
      out_type=jax.ShapeDtypeStruct((NC, nacc, D), jnp.float32),
      mesh=_mesh(),
      scratch_types=[
          pltpu.VMEM((3, G, ECH), jnp.int32),
          pltpu.VMEM((3, G, ECH), jnp.int32),
          pltpu.VMEM((ECH, D), jnp.float32),
          pltpu.VMEM((ECH, D), jnp.float32),
          pltpu.VMEM_SHARED((nacc, D), jnp.float32),
          pltpu.SemaphoreType.DMA,
          pltpu.SemaphoreType.DMA,
          pltpu.SemaphoreType.DMA,
          pltpu.SemaphoreType.DMA,
          pltpu.SemaphoreType.DMA,
      ],
  )
  def k(h_hbm, e_hbm, zrow_hbm, out_hbm,
        src_v, dst_v, rows_a, rows_b, acc_sh, sem_a, sem_b, ssa, ssb, isem):
    c = lax.axis_index("c")
    s = lax.axis_index("s")
    pltpu.sync_copy(e_hbm.at[0, c, s, pl.ds(0, G)], src_v.at[0])
    pltpu.sync_copy(e_hbm.at[1, c, s, pl.ds(0, G)], dst_v.at[0])
    pltpu.async_copy(e_hbm.at[0, c, s, pl.ds(G, G)], src_v.at[1], isem)
    pltpu.async_copy(e_hbm.at[1, c, s, pl.ds(G, G)], dst_v.at[1], isem)
    pltpu.sync_copy(zrow_hbm, rows_a)
    for kk in range(rpt // ECH):
      pltpu.sync_copy(rows_a, acc_sh.at[pl.ds(s * rpt + kk * ECH, ECH)])
    plsc.subcore_barrier()

    def wait(buf, sem):
      pltpu.make_async_copy(h_hbm.at[pl.ds(0, ECH)], buf, sem).wait()

    def src_row(j):
      return src_v.at[(j // G) % 3, j % G]

    def dst_row(j):
      return dst_v.at[(j // G) % 3, j % G]

    def fire(j, buf, sem):
      @pl.when(jnp.logical_and(j % G == 0, j > 0))
      def _():
        pltpu.make_async_copy(e_hbm.at[0, 0, 0, pl.ds(0, G)], src_v.at[0],
                              isem).wait()
        pltpu.make_async_copy(e_hbm.at[1, 0, 0, pl.ds(0, G)], dst_v.at[0],
                              isem).wait()
        st1 = j // G + 1

        @pl.when(st1 < nstg)
        def _():
          pltpu.async_copy(e_hbm.at[0, c, s, pl.ds(st1 * G, G)],
                           src_v.at[st1 % 3], isem)
          pltpu.async_copy(e_hbm.at[1, c, s, pl.ds(st1 * G, G)],
                           dst_v.at[st1 % 3], isem)

      pltpu.async_copy(h_hbm.at[src_row(j)], buf, sem)

    fire(0, rows_a, sem_a)

    def body(jj, carry):
      j0 = 2 * jj
      j1 = j0 + 1
      j2 = j0 + 2
      wait(rows_a, sem_a)
      fire(j1, rows_b, sem_b)
      pltpu.sync_copy(rows_a, acc_sh.at[dst_row(j0)], add=True)

      @pl.when(j2 < c_chunks)
      def _():
        fire(j2, rows_a, sem_a)

      wait(rows_b, sem_b)
      pltpu.sync_copy(rows_b, acc_sh.at[dst_row(j1)], add=True)
      return carry

    lax.fori_loop(0, c_chunks // 2, body, 0, unroll=False)
    plsc.subcore_barrier()
    pltpu.sync_copy(acc_sh.at[pl.ds(s * rpt, rpt)],
                    out_hbm.at[c, pl.ds(s * rpt, rpt)])

  return k


def _tc_w0(nacc):
  nb = nacc // TBLK

  def body(x, w, g_ref):
    g_ref[...] = jnp.dot(x[...], w[...], preferred_element_type=jnp.float32)

  return pl.pallas_call(
      body,
      grid=(nb,),
      in_specs=[
          pl.BlockSpec((TBLK, D), lambda i: (i, 0)),
          pl.BlockSpec((D, D), lambda i: (0, 0)),
      ],
      out_specs=pl.BlockSpec((TBLK, D), lambda i: (i, 0)),
      out_shape=jax.ShapeDtypeStruct((nacc, D), jnp.float32),
  )


def _tc_first(nacc):
  nb = nacc // TBLK

  def body(cnt, g, dinv_ref, h_ref):
    deg = cnt[0, :, 0:1] + cnt[1, :, 0:1] + 1.0
    dinv = lax.rsqrt(deg)
    dinv_ref[...] = dinv
    h_ref[...] = dinv * g[...]

  return pl.pallas_call(
      body,
      grid=(nb,),
      in_specs=[
          pl.BlockSpec((2, TBLK, DEGW), lambda i: (0, i, 0)),
          pl.BlockSpec((TBLK, D), lambda i: (i, 0)),
      ],
      out_specs=[
          pl.BlockSpec((TBLK, 1), lambda i: (i, 0)),
          pl.BlockSpec((TBLK, D), lambda i: (i, 0)),
      ],
      out_shape=[
          jax.ShapeDtypeStruct((nacc, 1), jnp.float32),
          jax.ShapeDtypeStruct((nacc, D), jnp.float32),
      ],
  )


def _tc_mid(nacc):
  nb = nacc // TBLK

  def body(a, hp, dinv, b, w, o_ref, h_ref):
    acc = a[0] + a[1] + hp[...]
    o = jnp.maximum(dinv[...] * acc + b[...], 0.0)
    o_ref[...] = o
    h_ref[...] = dinv[...] * jnp.dot(
        o, w[...], preferred_element_type=jnp.float32)

  return pl.pallas_call(
      body,
      grid=(nb,),
      in_specs=[
          pl.BlockSpec((2, TBLK, D), lambda i: (0, i, 0)),
          pl.BlockSpec((TBLK, D), lambda i: (i, 0)),
          pl.BlockSpec((TBLK, 1), lambda i: (i, 0)),
          pl.BlockSpec((1, D), lambda i: (0, 0)),
          pl.BlockSpec((D, D), lambda i: (0, 0)),
      ],
      out_specs=[
          pl.BlockSpec((TBLK, D), lambda i: (i, 0)),
          pl.BlockSpec((TBLK, D), lambda i: (i, 0)),
      ],
      out_shape=[
          jax.ShapeDtypeStruct((nacc, D), jnp.float32),
          jax.ShapeDtypeStruct((nacc, D), jnp.float32),
      ],
  )


def _tc_jk_partial(nacc):
  nb = nacc // TBLK

  def body(o0, o1, o2, wo, bo, out_ref):
    w = wo[...]
    r = jnp.dot(o0[...], w[0:128], preferred_element_type=jnp.float32)
    r += jnp.dot(o1[...], w[128:256], preferred_element_type=jnp.float32)
    r += jnp.dot(o2[...], w[256:384], preferred_element_type=jnp.float32)
    out_ref[...] = r + bo[...]

  return pl.pallas_call(
      body,
      grid=(nb,),
      in_specs=[
          pl.BlockSpec((TBLK, D), lambda i: (i, 0)),
          pl.BlockSpec((TBLK, D), lambda i: (i, 0)),
          pl.BlockSpec((TBLK, D), lambda i: (i, 0)),
          pl.BlockSpec((4 * D, D), lambda i: (0, 0)),
          pl.BlockSpec((1, D), lambda i: (0, 0)),
      ],
      out_specs=pl.BlockSpec((TBLK, D), lambda i: (i, 0)),
      out_shape=jax.ShapeDtypeStruct((nacc, D), jnp.float32),
  )


def _tc_last(nacc, n):
  nb = nacc // TBLK

  def body(a, hp, dinv, b, jkp, wo, out_ref):
    acc = a[0] + a[1] + hp[...]
    o3 = jnp.maximum(dinv[...] * acc + b[...], 0.0)
    w = wo[...]
    out_ref[...] = jkp[...] + jnp.dot(o3, w[384:512],
                                      preferred_element_type=jnp.float32)

  return pl.pallas_call(
      body,
      grid=(nb,),
      in_specs=[
          pl.BlockSpec((2, TBLK, D), lambda i: (0, i, 0)),
          pl.BlockSpec((TBLK, D), lambda i: (i, 0)),
          pl.BlockSpec((TBLK, 1), lambda i: (i, 0)),
          pl.BlockSpec((1, D), lambda i: (0, 0)),
          pl.BlockSpec((TBLK, D), lambda i: (i, 0)),
          pl.BlockSpec((4 * D, D), lambda i: (0, 0)),
      ],
      out_specs=pl.BlockSpec((TBLK, D), lambda i: (i, 0)),
      out_shape=jax.ShapeDtypeStruct((n, D), jnp.float32),
  )


def kernel(x, edge_index, W0, b0, W1, b1, W2, b2, W3, b3, Wout, bout):
  n = x.shape[0]
  e = edge_index.shape[1]

  nacc = ((n + NS * CHUNK - 1) // (NS * CHUNK)) * NS * CHUNK
  pad_rows = nacc - n

  ee = NW * ECH * G
  ep = ((e + ee - 1) // ee) * ee
  pad_e = ep - e
  c_chunks = ep // (NW * ECH)
  pad_idx = jnp.asarray(
      np.broadcast_to(n + (np.arange(pad_e) % pad_rows), (2, pad_e)),
      dtype=jnp.int32)
  edges = jnp.concatenate([edge_index, pad_idx], axis=1).reshape(
      2, NC, NS, c_chunks, ECH)

  ones_deg = jnp.ones((ECH, DEGW), jnp.float32)
  zeros_deg = jnp.zeros((nacc // NS, DEGW), jnp.float32)
  zrow = jnp.zeros((ECH, D), jnp.float32)
  x_pad = jnp.pad(x, ((0, pad_rows), (0, 0)))

  deg_k = _sc_degree(nacc, c_chunks)
  spmm_k = _sc_spmm(nacc, c_chunks)
  tc_w0 = _tc_w0(nacc)
  tc_first = _tc_first(nacc)
  tc_mid = _tc_mid(nacc)
  tc_jkp = _tc_jk_partial(nacc)
  tc_last = _tc_last(nacc, n)

  cnt = deg_k(edges, ones_deg, zeros_deg)
  g0 = tc_w0(x_pad, W0)
  dinv, hp = tc_first(cnt, g0)

  acc = spmm_k(hp, edges, zrow)
  o0, hp = tc_mid(acc, hp, dinv, b0.reshape(1, D), W1)
  acc = spmm_k(hp, edges, zrow)
  o1, hp = tc_mid(acc, hp, dinv, b1.reshape(1, D), W2)
  acc = spmm_k(hp, edges, zrow)
  o2, hp = tc_mid(acc, hp, dinv, b2.reshape(1, D), W3)
  acc = spmm_k(hp, edges, zrow)
  jkp = tc_jkp(o0, o1, o2, Wout, bout.reshape(1, D))
  out = tc_last(acc, hp, dinv, b3.reshape(1, D), jkp, Wout)
  return out
